# Initial kernel scaffold; baseline (speedup 1.0000x reference)
import jax
import jax.numpy as jnp
from jax.experimental import pallas as pl


def kernel(x, prot_edge_index, prot_edge_attr, edge_W, edge_b, pre_W1, pre_b1, pre_W2, pre_b2, post_W1, post_b1, post_W2, post_b2, lin_W, lin_b, ln_g, ln_b):
    raise NotImplementedError("write your pallas kernel here")


# speedup vs baseline: 15.7171x; 15.7171x over previous
import functools

import jax
import jax.numpy as jnp
import numpy as np
from jax import lax
from jax.experimental import pallas as pl

N = 10000
E = 160000
T = 4
F = 64
HID = 256
EDGE_DIM = 16

_DEG_HIST = np.array([0,1,2,5,11,23,44,79,135,216,324,457,605,753,880,966,997,966,880,753,605,457,324,216,135,79,44,23,11,5,2,1,0], dtype=np.float64)
_bins_ = np.arange(_DEG_HIST.shape[0], dtype=np.float64)
_AVG_LOG = float((np.log(_bins_ + 1.0) * _DEG_HIST).sum() / float(_DEG_HIST.sum()))

NB = 1000
EB = 2000


def _fold_kernel(edge_W_ref, edge_b_ref, pre_W1_ref, pre_b1_ref, cw_ref, cbias_ref):
    for t in range(T):
        w1c = pre_W1_ref[t, 128:192, :]
        cw_ref[:, t * F:(t + 1) * F] = jnp.dot(edge_W_ref[...], w1c,
                                               preferred_element_type=jnp.float32)
        cb = jnp.dot(edge_b_ref[...], w1c, preferred_element_type=jnp.float32)
        cbias_ref[0, t * F:(t + 1) * F] = cb + pre_b1_ref[t, :]


def _tables_kernel(x_ref, pre_W1_ref, a_ref, b_ref):
    for t in range(T):
        xt = x_ref[:, t * F:(t + 1) * F]
        a_ref[:, t * F:(t + 1) * F] = jnp.dot(xt, pre_W1_ref[t, 0:F, :],
                                              preferred_element_type=jnp.float32)
        b_ref[:, t * F:(t + 1) * F] = jnp.dot(xt, pre_W1_ref[t, F:2 * F, :],
                                              preferred_element_type=jnp.float32)


def _edge_kernel(g_ref, ea_ref, cw_ref, cbias_ref, w2_ref, b2_ref,
                 m0_ref, m1_ref, m2_ref, m3_ref):
    pre = g_ref[...] + jnp.dot(ea_ref[...], cw_ref[...],
                               preferred_element_type=jnp.float32) + cbias_ref[...]
    outs = (m0_ref, m1_ref, m2_ref, m3_ref)
    for t in range(T):
        mt = jnp.maximum(pre[:, t * F:(t + 1) * F], 0.0)
        outs[t][...] = (jnp.dot(mt, w2_ref[t], preferred_element_type=jnp.float32)
                        + b2_ref[t, :][None, :])


def _post_kernel(x_ref, s_ref, mn_ref, mx_ref, sq_ref, cnt_ref,
                 pw1_ref, pb1_ref, pw2_ref, pb2_ref, lw_ref, lb_ref,
                 lg_ref, lbeta_ref, out_ref):
    cnt = cnt_ref[...]
    deg = jnp.maximum(cnt, 1.0)
    inv_deg = 1.0 / deg
    has = (cnt > 0.0).astype(jnp.float32)
    lg1d = jnp.log(deg + 1.0)
    amp = lg1d * (1.0 / _AVG_LOG)
    att = _AVG_LOG / lg1d

    s = s_ref[...]
    mean = s * inv_deg
    mn = mn_ref[...] * has
    mx = mx_ref[...] * has
    msq = sq_ref[...] * inv_deg
    var = jnp.maximum(msq - mean * mean, 0.0)
    std = jnp.sqrt(var + 1e-5)

    o_parts = []
    for t in range(T):
        sl = slice(t * F, (t + 1) * F)
        aggt = jnp.concatenate([s[:, sl], mean[:, sl], mn[:, sl], mx[:, sl],
                                std[:, sl]], axis=1)
        w = pw1_ref[t]
        ot = jnp.dot(x_ref[:, sl], w[0:F, :], preferred_element_type=jnp.float32)
        ot = ot + jnp.dot(aggt, w[F:F + 5 * F, :], preferred_element_type=jnp.float32)
        ot = ot + amp * jnp.dot(aggt, w[F + 5 * F:F + 10 * F, :],
                                preferred_element_type=jnp.float32)
        ot = ot + att * jnp.dot(aggt, w[F + 10 * F:F + 15 * F, :],
                                preferred_element_type=jnp.float32)
        ot = jnp.maximum(ot + pb1_ref[t, :][None, :], 0.0)
        ot = jnp.dot(ot, pw2_ref[t], preferred_element_type=jnp.float32) + pb2_ref[t, :][None, :]
        o_parts.append(ot)
    o = jnp.concatenate(o_parts, axis=1)

    o = jnp.dot(o, lw_ref[...], preferred_element_type=jnp.float32) + lb_ref[0, :][None, :]
    mu = jnp.mean(o, axis=1, keepdims=True)
    vv = jnp.mean((o - mu) ** 2, axis=1, keepdims=True)
    ln = (o - mu) * lax.rsqrt(vv + 1e-5) * lg_ref[0, :][None, :] + lbeta_ref[0, :][None, :]
    out_ref[...] = x_ref[...] + jnp.maximum(ln, 0.0)


def kernel(x, prot_edge_index, prot_edge_attr, edge_W, edge_b, pre_W1, pre_b1,
           pre_W2, pre_b2, post_W1, post_b1, post_W2, post_b2, lin_W, lin_b,
           ln_g, ln_b):
    src = prot_edge_index[0]
    dst = prot_edge_index[1]

    cw, cbias = pl.pallas_call(
        _fold_kernel,
        out_shape=[jax.ShapeDtypeStruct((EDGE_DIM, HID), jnp.float32),
                   jax.ShapeDtypeStruct((1, HID), jnp.float32)],
    )(edge_W, edge_b, pre_W1, pre_b1)

    a_tab, b_tab = pl.pallas_call(
        _tables_kernel,
        grid=(N // NB,),
        in_specs=[pl.BlockSpec((NB, HID), lambda i: (i, 0)),
                  pl.BlockSpec((T, 3 * F, F), lambda i: (0, 0, 0))],
        out_specs=[pl.BlockSpec((NB, HID), lambda i: (i, 0)),
                   pl.BlockSpec((NB, HID), lambda i: (i, 0))],
        out_shape=[jax.ShapeDtypeStruct((N, HID), jnp.float32),
                   jax.ShapeDtypeStruct((N, HID), jnp.float32)],
    )(x, pre_W1)

    g = a_tab[dst] + b_tab[src]

    mcols = pl.pallas_call(
        _edge_kernel,
        grid=(E // EB,),
        in_specs=[pl.BlockSpec((EB, HID), lambda i: (i, 0)),
                  pl.BlockSpec((EB, EDGE_DIM), lambda i: (i, 0)),
                  pl.BlockSpec((EDGE_DIM, HID), lambda i: (0, 0)),
                  pl.BlockSpec((1, HID), lambda i: (0, 0)),
                  pl.BlockSpec((T, F, F), lambda i: (0, 0, 0)),
                  pl.BlockSpec((T, F), lambda i: (0, 0))],
        out_specs=[pl.BlockSpec((EB, F), lambda i: (i, 0))] * T,
        out_shape=[jax.ShapeDtypeStruct((E, F), jnp.float32)] * T,
    )(g, prot_edge_attr, cw, cbias, pre_W2, pre_b2)

    m = jnp.concatenate(mcols, axis=1)
    ones = jnp.ones((E,), jnp.float32)
    cnt = jax.ops.segment_sum(ones, dst, num_segments=N)
    s = jax.ops.segment_sum(m, dst, num_segments=N)
    sq = jax.ops.segment_sum(m * m, dst, num_segments=N)
    mn = jax.ops.segment_min(m, dst, num_segments=N)
    mn = jnp.where(jnp.isfinite(mn), mn, 0.0)
    mx = jax.ops.segment_max(m, dst, num_segments=N)
    mx = jnp.where(jnp.isfinite(mx), mx, 0.0)

    out = pl.pallas_call(
        _post_kernel,
        grid=(N // NB,),
        in_specs=[pl.BlockSpec((NB, HID), lambda i: (i, 0)),
                  pl.BlockSpec((NB, HID), lambda i: (i, 0)),
                  pl.BlockSpec((NB, HID), lambda i: (i, 0)),
                  pl.BlockSpec((NB, HID), lambda i: (i, 0)),
                  pl.BlockSpec((NB, HID), lambda i: (i, 0)),
                  pl.BlockSpec((NB, 1), lambda i: (i, 0)),
                  pl.BlockSpec((T, 16 * F, F), lambda i: (0, 0, 0)),
                  pl.BlockSpec((T, F), lambda i: (0, 0)),
                  pl.BlockSpec((T, F, F), lambda i: (0, 0, 0)),
                  pl.BlockSpec((T, F), lambda i: (0, 0)),
                  pl.BlockSpec((HID, HID), lambda i: (0, 0)),
                  pl.BlockSpec((1, HID), lambda i: (0, 0)),
                  pl.BlockSpec((1, HID), lambda i: (0, 0)),
                  pl.BlockSpec((1, HID), lambda i: (0, 0))],
        out_specs=pl.BlockSpec((NB, HID), lambda i: (i, 0)),
        out_shape=jax.ShapeDtypeStruct((N, HID), jnp.float32),
    )(x, s, mn, mx, sq, cnt.reshape(N, 1), post_W1, post_b1, post_W2,
      post_b2, lin_W, lin_b.reshape(1, HID), ln_g.reshape(1, HID),
      ln_b.reshape(1, HID))
    return out
